# baseline (speedup 1.0000x reference)
import functools

import jax
import jax.numpy as jnp
from jax import lax
from jax.experimental import pallas as pl
from jax.experimental.pallas import tpu as pltpu
from jax.experimental.pallas import tpu_sc as plsc

_N = 10000
_E = 320000
_D = 128
_DOUT = 51
_LMP = 5

_NC = 2
_NS = 16
_NW = _NC * _NS
_EPW = _E // _NW
_CH = 40
_GR = 10
_NG = _EPW // (_GR * _CH)
_RPT = 640
_NPAD = _RPT * _NS
_LANES = 16


def _sc_edge_body(kqv_hbm, gidx_hbm, didx_hbm, zeros_hbm, out_hbm,
                  agg_s, gidx_v, didx_v, tb0, tb1, gsem0, gsem1):
    cid = lax.axis_index("c")
    sid = lax.axis_index("s")
    wid = sid * _NC + cid
    bufs = ((tb0, gsem0), (tb1, gsem1))
    dummy = kqv_hbm.at[pl.ds(0, 3 * _CH)]

    pltpu.sync_copy(zeros_hbm, agg_s.at[pl.ds(sid * _RPT, _RPT)])
    plsc.subcore_barrier()

    def fire(c, b):
        tb, gsem = bufs[b]
        pltpu.async_copy(kqv_hbm.at[gidx_v.at[c]], tb, gsem)

    def finish(c, b):
        tb, gsem = bufs[b]
        pltpu.make_async_copy(dummy, tb, gsem).wait()

        def row(e2, carry):
            for r in range(2):
                e = 2 * e2 + r
                for j in range(_D // _LANES):
                    sl = pl.ds(j * _LANES, _LANES)
                    gate = 1.0 / (1.0 + jnp.exp(-(tb[e, sl] + tb[_CH + e, sl])))
                    tb[2 * _CH + e, sl] = gate * tb[2 * _CH + e, sl]
            return carry

        lax.fori_loop(0, _CH // 2, row, 0)
        pltpu.sync_copy(tb.at[pl.ds(2 * _CH, _CH)], agg_s.at[didx_v.at[c]],
                        add=True)

    def group(g, carry):
        pltpu.sync_copy(gidx_hbm.at[wid, g], gidx_v)
        pltpu.sync_copy(didx_hbm.at[wid, g], didx_v)
        fire(0, 0)

        def pair(t, carry2):
            fire(2 * t + 1, 1)
            finish(2 * t, 0)
            pl.when(t < _GR // 2 - 1)(lambda: fire(2 * t + 2, 0))
            finish(2 * t + 1, 1)
            return carry2

        lax.fori_loop(0, _GR // 2, pair, 0)
        return carry

    lax.fori_loop(0, _NG, group, 0)
    plsc.subcore_barrier()
    pltpu.sync_copy(agg_s.at[pl.ds(sid * _RPT, _RPT)],
                    out_hbm.at[cid, pl.ds(sid * _RPT, _RPT)])


@jax.jit
def _sc_edge_call(kqv, gidx, didx, zeros):
    mesh = plsc.VectorSubcoreMesh(core_axis_name="c", subcore_axis_name="s",
                                  num_cores=_NC, num_subcores=_NS)
    return pl.kernel(
        _sc_edge_body,
        out_type=jax.ShapeDtypeStruct((_NC, _NPAD, _D), jnp.float32),
        mesh=mesh,
        scratch_types=[
            pltpu.MemorySpace.VMEM_SHARED((_NPAD, _D), jnp.float32),
            pltpu.VMEM((_GR, 3 * _CH), jnp.int32),
            pltpu.VMEM((_GR, _CH), jnp.int32),
            pltpu.VMEM((3 * _CH, _D), jnp.float32),
            pltpu.VMEM((3 * _CH, _D), jnp.float32),
            pltpu.SemaphoreType.DMA,
            pltpu.SemaphoreType.DMA,
        ],
    )(kqv, gidx, didx, zeros)


def _bn(h, g, b):
    mu = jnp.mean(h, axis=0, keepdims=True)
    var = jnp.mean((h - mu) * (h - mu), axis=0, keepdims=True)
    return g * (h - mu) / jnp.sqrt(var + 1e-5) + b


def _l2n(h):
    return h / (jnp.sqrt(jnp.sum(h * h, axis=-1, keepdims=True)) + 1e-12)


def _pre_body(x_ref, wp_ref, bp_ref, gp_ref, bep_ref, wk_ref, wq_ref, wv_ref,
              h_ref, t_ref):
    h = jnp.dot(x_ref[...], wp_ref[...], preferred_element_type=jnp.float32)
    h = _l2n(jnp.maximum(_bn(h + bp_ref[...], gp_ref[...], bep_ref[...]), 0.0))
    h_ref[...] = h
    t_ref[0:_N] = jnp.dot(h, wk_ref[...], preferred_element_type=jnp.float32)
    t_ref[_N:2 * _N] = jnp.dot(h, wq_ref[...], preferred_element_type=jnp.float32)
    t_ref[2 * _N:3 * _N] = jnp.dot(h, wv_ref[...], preferred_element_type=jnp.float32)


@jax.jit
def _pre_call(x, wp, bp, gp, bep, wk, wq, wv):
    f = jax.ShapeDtypeStruct
    return pl.pallas_call(
        _pre_body,
        out_shape=(f((_N, _D), jnp.float32), f((3 * _N, _D), jnp.float32)),
    )(x, wp, bp, gp, bep, wk, wq, wv)


def _mid_body(h_ref, agg_ref, ws_ref, b_ref, g_ref, be_ref,
              wk_ref, wq_ref, wv_ref, hn_ref, t_ref):
    h = h_ref[...]
    out = (agg_ref[0, :_N] + agg_ref[1, :_N]
           + jnp.dot(h, ws_ref[...], preferred_element_type=jnp.float32)
           + b_ref[...])
    out = _l2n(jnp.maximum(_bn(out, g_ref[...], be_ref[...]), 0.0))
    hn = h + out
    hn_ref[...] = hn
    t_ref[0:_N] = jnp.dot(hn, wk_ref[...], preferred_element_type=jnp.float32)
    t_ref[_N:2 * _N] = jnp.dot(hn, wq_ref[...], preferred_element_type=jnp.float32)
    t_ref[2 * _N:3 * _N] = jnp.dot(hn, wv_ref[...], preferred_element_type=jnp.float32)


@jax.jit
def _mid_call(h, aggs, ws, b, g, be, wk, wq, wv):
    f = jax.ShapeDtypeStruct
    return pl.pallas_call(
        _mid_body,
        out_shape=(f((_N, _D), jnp.float32), f((3 * _N, _D), jnp.float32)),
    )(h, aggs, ws, b, g, be, wk, wq, wv)


def _last_body(h_ref, agg_ref, ws_ref, b_ref, g_ref, be_ref,
               w1_ref, b1_ref, g1_ref, be1_ref, w2_ref, b2_ref, o_ref):
    h = h_ref[...]
    out = (agg_ref[0, :_N] + agg_ref[1, :_N]
           + jnp.dot(h, ws_ref[...], preferred_element_type=jnp.float32)
           + b_ref[...])
    out = _l2n(jnp.maximum(_bn(out, g_ref[...], be_ref[...]), 0.0))
    h = _l2n(h + out)
    h = jnp.dot(h, w1_ref[...], preferred_element_type=jnp.float32) + b1_ref[...]
    h = _l2n(jnp.maximum(_bn(h, g1_ref[...], be1_ref[...]), 0.0))
    z = jnp.dot(h, w2_ref[...], preferred_element_type=jnp.float32) + b2_ref[...]
    o_ref[...] = 1.0 / (1.0 + jnp.exp(-z))


@jax.jit
def _last_call(h, aggs, ws, b, g, be, w1, b1, g1, be1, w2, b2):
    return pl.pallas_call(
        _last_body,
        out_shape=jax.ShapeDtypeStruct((_N, _DOUT), jnp.float32),
    )(h, aggs, ws, b, g, be, w1, b1, g1, be1, w2, b2)


def kernel(x, edge_index, W_pre, b_pre, g_pre, be_pre, Wk, Wq, Wv, Ws,
           b_mp, g_mp, be_mp, W1, b1, g1, be1, W2, b2):
    src = edge_index[0].astype(jnp.int32).reshape(_NW, _NG, _GR, 1, _CH)
    dst = edge_index[1].astype(jnp.int32).reshape(_NW, _NG, _GR, 1, _CH)
    gidx = jnp.concatenate([dst, src + _N, src + 2 * _N],
                           axis=3).reshape(_NW, _NG, _GR, 3 * _CH)
    didx = dst.reshape(_NW, _NG, _GR, _CH)
    zeros = jnp.zeros((_RPT, _D), jnp.float32)
    r = lambda a: a.reshape(1, -1)

    h, kqv = _pre_call(x, W_pre, r(b_pre), r(g_pre), r(be_pre),
                       Wk[0], Wq[0], Wv[0])
    for i in range(_LMP):
        aggs = _sc_edge_call(kqv, gidx, didx, zeros)
        if i < _LMP - 1:
            h, kqv = _mid_call(h, aggs, Ws[i], r(b_mp[i]), r(g_mp[i]),
                               r(be_mp[i]), Wk[i + 1], Wq[i + 1], Wv[i + 1])
        else:
            out = _last_call(h, aggs, Ws[i], r(b_mp[i]), r(g_mp[i]),
                             r(be_mp[i]), W1, r(b1), r(g1), r(be1), W2, r(b2))
    return out


# speedup vs baseline: 1.2704x; 1.2704x over previous
import functools

import jax
import jax.numpy as jnp
from jax import lax
from jax.experimental import pallas as pl
from jax.experimental.pallas import tpu as pltpu
from jax.experimental.pallas import tpu_sc as plsc

_N = 10000
_E = 320000
_D = 128
_DOUT = 51
_LMP = 5

_NC = 2
_NS = 16
_NW = _NC * _NS
_EPW = _E // _NW
_CH = 40
_GR = 50
_NG = _EPW // (_GR * _CH)
_RPT = 640
_NPAD = _RPT * _NS
_LANES = 16


def _sc_edge_body(kqv_hbm, gidx_hbm, zeros_hbm, out_hbm,
                  agg_s, gidx_v, tb0, tb1, mb0, mb1, gsem0, gsem1,
                  ssem0, ssem1):
    cid = lax.axis_index("c")
    sid = lax.axis_index("s")
    wid = sid * _NC + cid
    bufs = ((tb0, mb0, gsem0, ssem0), (tb1, mb1, gsem1, ssem1))
    dummy = kqv_hbm.at[pl.ds(0, 3 * _CH)]
    mdummy = kqv_hbm.at[pl.ds(0, _CH)]

    pltpu.sync_copy(zeros_hbm, agg_s.at[pl.ds(sid * _RPT, _RPT)])
    plsc.subcore_barrier()

    def fire(c, b):
        tb, mb, gsem, ssem = bufs[b]
        pltpu.async_copy(kqv_hbm.at[gidx_v.at[c]], tb, gsem)

    def drain_scatter(b):
        tb, mb, gsem, ssem = bufs[b]
        pltpu.make_async_copy(mdummy, mb, ssem).wait()

    def finish(c, b, not_first):
        tb, mb, gsem, ssem = bufs[b]
        pltpu.make_async_copy(dummy, tb, gsem).wait()
        pl.when(not_first)(lambda: drain_scatter(b))

        def row(e2, carry):
            for r in range(2):
                e = 2 * e2 + r
                for j in range(_D // _LANES):
                    sl = pl.ds(j * _LANES, _LANES)
                    gate = 1.0 / (1.0 + jnp.exp(-(tb[e, sl] + tb[_CH + e, sl])))
                    mb[e, sl] = gate * tb[2 * _CH + e, sl]
            return carry

        lax.fori_loop(0, _CH // 2, row, 0)
        pltpu.async_copy(mb, agg_s.at[gidx_v.at[c, pl.ds(0, _CH)]], ssem,
                         add=True)

    def group(g, carry):
        @pl.when(g > 0)
        def _():
            drain_scatter(0)
            drain_scatter(1)
        pltpu.sync_copy(gidx_hbm.at[wid, g], gidx_v)
        fire(0, 0)

        def pair(t, carry2):
            fire(2 * t + 1, 1)
            finish(2 * t, 0, t > 0)
            pl.when(t < _GR // 2 - 1)(lambda: fire(2 * t + 2, 0))
            finish(2 * t + 1, 1, t > 0)
            return carry2

        lax.fori_loop(0, _GR // 2, pair, 0)
        return carry

    lax.fori_loop(0, _NG, group, 0)
    drain_scatter(0)
    drain_scatter(1)
    plsc.subcore_barrier()
    pltpu.sync_copy(agg_s.at[pl.ds(sid * _RPT, _RPT)],
                    out_hbm.at[cid, pl.ds(sid * _RPT, _RPT)])


@jax.jit
def _sc_edge_call(kqv, gidx, zeros):
    mesh = plsc.VectorSubcoreMesh(core_axis_name="c", subcore_axis_name="s",
                                  num_cores=_NC, num_subcores=_NS)
    return pl.kernel(
        _sc_edge_body,
        out_type=jax.ShapeDtypeStruct((_NC, _NPAD, _D), jnp.float32),
        mesh=mesh,
        scratch_types=[
            pltpu.MemorySpace.VMEM_SHARED((_NPAD, _D), jnp.float32),
            pltpu.VMEM((_GR, 3 * _CH), jnp.int32),
            pltpu.VMEM((3 * _CH, _D), jnp.float32),
            pltpu.VMEM((3 * _CH, _D), jnp.float32),
            pltpu.VMEM((_CH, _D), jnp.float32),
            pltpu.VMEM((_CH, _D), jnp.float32),
            pltpu.SemaphoreType.DMA,
            pltpu.SemaphoreType.DMA,
            pltpu.SemaphoreType.DMA,
            pltpu.SemaphoreType.DMA,
        ],
    )(kqv, gidx, zeros)


def _bn(h, g, b):
    mu = jnp.mean(h, axis=0, keepdims=True)
    var = jnp.mean((h - mu) * (h - mu), axis=0, keepdims=True)
    return g * (h - mu) / jnp.sqrt(var + 1e-5) + b


def _l2n(h):
    return h / (jnp.sqrt(jnp.sum(h * h, axis=-1, keepdims=True)) + 1e-12)


def _pre_body(x_ref, wp_ref, bp_ref, gp_ref, bep_ref, wk_ref, wq_ref, wv_ref,
              h_ref, t_ref):
    h = jnp.dot(x_ref[...], wp_ref[...], preferred_element_type=jnp.float32)
    h = _l2n(jnp.maximum(_bn(h + bp_ref[...], gp_ref[...], bep_ref[...]), 0.0))
    h_ref[...] = h
    t_ref[0:_N] = jnp.dot(h, wk_ref[...], preferred_element_type=jnp.float32)
    t_ref[_N:2 * _N] = jnp.dot(h, wq_ref[...], preferred_element_type=jnp.float32)
    t_ref[2 * _N:3 * _N] = jnp.dot(h, wv_ref[...], preferred_element_type=jnp.float32)


@jax.jit
def _pre_call(x, wp, bp, gp, bep, wk, wq, wv):
    f = jax.ShapeDtypeStruct
    return pl.pallas_call(
        _pre_body,
        out_shape=(f((_N, _D), jnp.float32), f((3 * _N, _D), jnp.float32)),
    )(x, wp, bp, gp, bep, wk, wq, wv)


def _mid_body(h_ref, agg_ref, ws_ref, b_ref, g_ref, be_ref,
              wk_ref, wq_ref, wv_ref, hn_ref, t_ref):
    h = h_ref[...]
    out = (agg_ref[0, :_N] + agg_ref[1, :_N]
           + jnp.dot(h, ws_ref[...], preferred_element_type=jnp.float32)
           + b_ref[...])
    out = _l2n(jnp.maximum(_bn(out, g_ref[...], be_ref[...]), 0.0))
    hn = h + out
    hn_ref[...] = hn
    t_ref[0:_N] = jnp.dot(hn, wk_ref[...], preferred_element_type=jnp.float32)
    t_ref[_N:2 * _N] = jnp.dot(hn, wq_ref[...], preferred_element_type=jnp.float32)
    t_ref[2 * _N:3 * _N] = jnp.dot(hn, wv_ref[...], preferred_element_type=jnp.float32)


@jax.jit
def _mid_call(h, aggs, ws, b, g, be, wk, wq, wv):
    f = jax.ShapeDtypeStruct
    return pl.pallas_call(
        _mid_body,
        out_shape=(f((_N, _D), jnp.float32), f((3 * _N, _D), jnp.float32)),
    )(h, aggs, ws, b, g, be, wk, wq, wv)


def _last_body(h_ref, agg_ref, ws_ref, b_ref, g_ref, be_ref,
               w1_ref, b1_ref, g1_ref, be1_ref, w2_ref, b2_ref, o_ref):
    h = h_ref[...]
    out = (agg_ref[0, :_N] + agg_ref[1, :_N]
           + jnp.dot(h, ws_ref[...], preferred_element_type=jnp.float32)
           + b_ref[...])
    out = _l2n(jnp.maximum(_bn(out, g_ref[...], be_ref[...]), 0.0))
    h = _l2n(h + out)
    h = jnp.dot(h, w1_ref[...], preferred_element_type=jnp.float32) + b1_ref[...]
    h = _l2n(jnp.maximum(_bn(h, g1_ref[...], be1_ref[...]), 0.0))
    z = jnp.dot(h, w2_ref[...], preferred_element_type=jnp.float32) + b2_ref[...]
    o_ref[...] = 1.0 / (1.0 + jnp.exp(-z))


@jax.jit
def _last_call(h, aggs, ws, b, g, be, w1, b1, g1, be1, w2, b2):
    return pl.pallas_call(
        _last_body,
        out_shape=jax.ShapeDtypeStruct((_N, _DOUT), jnp.float32),
    )(h, aggs, ws, b, g, be, w1, b1, g1, be1, w2, b2)


def kernel(x, edge_index, W_pre, b_pre, g_pre, be_pre, Wk, Wq, Wv, Ws,
           b_mp, g_mp, be_mp, W1, b1, g1, be1, W2, b2):
    src = edge_index[0].astype(jnp.int32).reshape(_NW, _NG, _GR, 1, _CH)
    dst = edge_index[1].astype(jnp.int32).reshape(_NW, _NG, _GR, 1, _CH)
    gidx = jnp.concatenate([dst, src + _N, src + 2 * _N],
                           axis=3).reshape(_NW, _NG, _GR, 3 * _CH)
    zeros = jnp.zeros((_RPT, _D), jnp.float32)
    r = lambda a: a.reshape(1, -1)

    h, kqv = _pre_call(x, W_pre, r(b_pre), r(g_pre), r(be_pre),
                       Wk[0], Wq[0], Wv[0])
    for i in range(_LMP):
        aggs = _sc_edge_call(kqv, gidx, zeros)
        if i < _LMP - 1:
            h, kqv = _mid_call(h, aggs, Ws[i], r(b_mp[i]), r(g_mp[i]),
                               r(be_mp[i]), Wk[i + 1], Wq[i + 1], Wv[i + 1])
        else:
            out = _last_call(h, aggs, Ws[i], r(b_mp[i]), r(g_mp[i]),
                             r(be_mp[i]), W1, r(b1), r(g1), r(be1), W2, r(b2))
    return out
